# Initial kernel scaffold; baseline (speedup 1.0000x reference)
import jax
import jax.numpy as jnp
from jax.experimental import pallas as pl


def kernel(user, item, history, length, user_table, item_table, cate_table, cate_list, bn_gamma, bn_beta, W1, b1, a1, W2, b2, a2, W3, b3):
    raise NotImplementedError("write your pallas kernel here")


# speedup vs baseline: 18.5894x; 18.5894x over previous
import functools

import jax
import jax.numpy as jnp
from jax import lax
from jax.experimental import pallas as pl
from jax.experimental.pallas import tpu as pltpu
from jax.experimental.pallas import tpu_sc as plsc

B = 16384
L = 20
D = 128
NC = 2
NS = 16
NW = NC * NS
RPW = B // NW
UI_CHUNK = 128
H_CHUNK = 32
N_HCHUNK = RPW // H_CHUNK


def _sc_body(uidx_hbm, iidx_hbm, hidx_hbm, utab, ctab, uout, iout, hout,
             uidx_v, iidx_v, hidx_v, rbuf, hbuf, habuf, sem):
    c = lax.axis_index("c")
    s = lax.axis_index("s")
    wid = c * NS + s
    base = wid * RPW

    pltpu.sync_copy(uidx_hbm.at[pl.ds(base, RPW)], uidx_v)
    pltpu.sync_copy(iidx_hbm.at[pl.ds(base, RPW)], iidx_v)
    pltpu.sync_copy(hidx_hbm.at[pl.ds(base * L, RPW * L)], hidx_v)

    def ui_chunk(k, _):
        cp = pltpu.async_copy(utab.at[uidx_v.at[pl.ds(k * UI_CHUNK, UI_CHUNK)]], rbuf, sem)
        cp.wait()
        pltpu.sync_copy(rbuf, uout.at[pl.ds(base + k * UI_CHUNK, UI_CHUNK)])
        return 0
    lax.fori_loop(0, RPW // UI_CHUNK, ui_chunk, 0)

    def it_chunk(k, _):
        cp = pltpu.async_copy(ctab.at[iidx_v.at[pl.ds(k * UI_CHUNK, UI_CHUNK)]], rbuf, sem)
        cp.wait()
        pltpu.sync_copy(rbuf, iout.at[pl.ds(base + k * UI_CHUNK, UI_CHUNK)])
        return 0
    lax.fori_loop(0, RPW // UI_CHUNK, it_chunk, 0)

    def hist_chunk(ci, _):
        cps = []
        for j in range(5):
            cps.append(pltpu.async_copy(
                ctab.at[hidx_v.at[pl.ds(ci * (H_CHUNK * L) + j * 128, 128)]],
                hbuf.at[pl.ds(j * 128, 128)], sem))
        for cp in cps:
            cp.wait()

        def row(r, _):
            rb = r * L
            for col in range(D // 16):
                cs = pl.ds(col * 16, 16)
                acc = hbuf[rb, cs]
                for l in range(1, L):
                    acc = acc + hbuf[rb + l, cs]
                habuf[r, cs] = acc
            return 0
        lax.fori_loop(0, H_CHUNK, row, 0)
        pltpu.sync_copy(habuf, hout.at[pl.ds(base + ci * H_CHUNK, H_CHUNK)])
        return 0
    lax.fori_loop(0, N_HCHUNK, hist_chunk, 0)


@jax.jit
def _sc_gather(uidx, iidx, hidx, utab, ctab):
    mesh = plsc.VectorSubcoreMesh(core_axis_name="c", subcore_axis_name="s")
    f = pl.kernel(
        _sc_body,
        out_type=[jax.ShapeDtypeStruct((B, D), jnp.float32)] * 3,
        mesh=mesh,
        scratch_types=[
            pltpu.VMEM((RPW,), jnp.int32),
            pltpu.VMEM((RPW,), jnp.int32),
            pltpu.VMEM((RPW * L,), jnp.int32),
            pltpu.VMEM((UI_CHUNK, D), jnp.float32),
            pltpu.VMEM((H_CHUNK * L, D), jnp.float32),
            pltpu.VMEM((H_CHUNK, D), jnp.float32),
            pltpu.SemaphoreType.DMA,
        ],
    )
    return f(uidx, iidx, hidx, utab, ctab)


BLK = 2048


def _stats_body(u_ref, i_ref, h_ref, s_out, q_out, acc_s, acc_q):
    @pl.when(pl.program_id(0) == 0)
    def _():
        acc_s[...] = jnp.zeros_like(acc_s)
        acc_q[...] = jnp.zeros_like(acc_q)

    x = jnp.concatenate([u_ref[...], i_ref[...], h_ref[...]], axis=1)
    acc_s[...] += jnp.sum(x, axis=0, keepdims=True)
    acc_q[...] += jnp.sum(x * x, axis=0, keepdims=True)

    @pl.when(pl.program_id(0) == pl.num_programs(0) - 1)
    def _():
        s_out[...] = acc_s[...]
        q_out[...] = acc_q[...]


def _apply_body(u_ref, i_ref, h_ref, s_ref, q_ref, g_ref, be_ref,
                w1_ref, b1_ref, a1_ref, w2_ref, b2_ref, a2_ref,
                w3_ref, b3_ref, o_ref):
    inv_b = 1.0 / B
    mean = s_ref[...] * inv_b
    var = q_ref[...] * inv_b - mean * mean
    inv = lax.rsqrt(var + 1e-5)
    scale = g_ref[...] * inv
    shift = be_ref[...] - mean * scale
    x = jnp.concatenate([u_ref[...], i_ref[...], h_ref[...]], axis=1)
    xh = x * scale + shift
    a1 = a1_ref[0, 0]
    h1 = jnp.dot(xh, w1_ref[...], preferred_element_type=jnp.float32) + b1_ref[...]
    h1 = jnp.maximum(h1, 0.0) + a1 * jnp.minimum(h1, 0.0)
    a2 = a2_ref[0, 0]
    h2 = jnp.dot(h1, w2_ref[...], preferred_element_type=jnp.float32) + b2_ref[...]
    h2 = jnp.maximum(h2, 0.0) + a2 * jnp.minimum(h2, 0.0)
    lg = jnp.dot(h2, w3_ref[...], preferred_element_type=jnp.float32) + b3_ref[...]
    m = jnp.max(lg, axis=1, keepdims=True)
    e = jnp.exp(lg - m)
    o_ref[...] = e / jnp.sum(e, axis=1, keepdims=True)


def _tc_head(u_rows, i_rows, h_rows, bn_gamma, bn_beta, W1, b1, a1, W2, b2, a2, W3, b3):
    nblk = B // BLK
    blk = lambda j: pl.BlockSpec((BLK, D), lambda i: (i, 0))
    full = lambda r, c: pl.BlockSpec((r, c), lambda i: (0, 0))
    sums, sqs = pl.pallas_call(
        _stats_body,
        grid=(nblk,),
        in_specs=[blk(0), blk(1), blk(2)],
        out_specs=[full(1, 384), full(1, 384)],
        out_shape=[jax.ShapeDtypeStruct((1, 384), jnp.float32)] * 2,
        scratch_shapes=[pltpu.VMEM((1, 384), jnp.float32)] * 2,
    )(u_rows, i_rows, h_rows)

    probs = pl.pallas_call(
        _apply_body,
        grid=(nblk,),
        in_specs=[blk(0), blk(1), blk(2),
                  full(1, 384), full(1, 384), full(1, 384), full(1, 384),
                  full(384, 200), full(1, 200), full(1, 1),
                  full(200, 80), full(1, 80), full(1, 1),
                  full(80, 2), full(1, 2)],
        out_specs=pl.BlockSpec((BLK, 2), lambda i: (i, 0)),
        out_shape=jax.ShapeDtypeStruct((B, 2), jnp.float32),
    )(u_rows, i_rows, h_rows, sums, sqs,
      bn_gamma.reshape(1, 384), bn_beta.reshape(1, 384),
      W1, b1.reshape(1, 200), a1.reshape(1, 1),
      W2, b2.reshape(1, 80), a2.reshape(1, 1),
      W3, b3.reshape(1, 2))
    return probs


def kernel(user, item, history, length, user_table, item_table, cate_table,
           cate_list, bn_gamma, bn_beta, W1, b1, a1, W2, b2, a2, W3, b3):
    ctab = jnp.concatenate(
        [item_table, jnp.take(cate_table, cate_list, axis=0)], axis=1)
    u_rows, i_rows, h_rows = _sc_gather(
        user.astype(jnp.int32), item.astype(jnp.int32),
        history.reshape(-1).astype(jnp.int32), user_table, ctab)
    return _tc_head(u_rows, i_rows, h_rows, bn_gamma, bn_beta,
                    W1, b1, a1, W2, b2, a2, W3, b3)


# speedup vs baseline: 25.3281x; 1.3625x over previous
import jax
import jax.numpy as jnp
from jax import lax
from jax.experimental import pallas as pl
from jax.experimental.pallas import tpu as pltpu
from jax.experimental.pallas import tpu_sc as plsc

B = 16384
L = 20
D = 128
NC = 2
NS = 16
NW = NC * NS
RPW = B // NW
UI_CHUNK = 128
HC = 16
NHC = RPW // HC
HROWS = HC * L


def _sc_body(uidx_hbm, iidx_hbm, hidx_hbm, utab, ctab, uout, iout, hout,
             uidx_v, iidx_v, hidx_v, rbuf, hbuf, habuf,
             gsem0, gsem1, osem0, osem1, usem0, usem1, wsem0, wsem1):
    c = lax.axis_index("c")
    s = lax.axis_index("s")
    wid = c * NS + s
    base = wid * RPW

    cp_u = pltpu.async_copy(uidx_hbm.at[pl.ds(base, RPW)], uidx_v, usem0)
    cp_i = pltpu.async_copy(iidx_hbm.at[pl.ds(base, RPW)], iidx_v, usem1)
    cp_h = pltpu.async_copy(hidx_hbm.at[pl.ds(base * L, RPW * L)], hidx_v, gsem0)
    cp_u.wait()
    cp_i.wait()
    cp_h.wait()

    plan = ([(utab, uidx_v, uout, k) for k in range(RPW // UI_CHUNK)] +
            [(ctab, iidx_v, iout, k) for k in range(RPW // UI_CHUNK)])
    gse = [usem0, usem1]
    wse = [wsem0, wsem1]

    def fire_ui(t):
        tab, idxv, _, k = plan[t]
        return pltpu.async_copy(
            tab.at[idxv.at[pl.ds(k * UI_CHUNK, UI_CHUNK)]], rbuf.at[t % 2],
            gse[t % 2])

    hs = {0: fire_ui(0)}
    outs = {}
    for t in range(len(plan)):
        if t + 1 < len(plan):
            if t - 1 >= 0:
                outs[t - 1].wait()
            hs[t + 1] = fire_ui(t + 1)
        hs[t].wait()
        _, _, out, k = plan[t]
        outs[t] = pltpu.async_copy(
            rbuf.at[t % 2], out.at[pl.ds(base + k * UI_CHUNK, UI_CHUNK)],
            wse[t % 2])
    outs[len(plan) - 2].wait()
    outs[len(plan) - 1].wait()

    def fire_h(ci, buf, sem):
        c0 = ci * HROWS
        pltpu.async_copy(ctab.at[hidx_v.at[pl.ds(c0, 128)]],
                         buf.at[pl.ds(0, 128)], sem)
        pltpu.async_copy(ctab.at[hidx_v.at[pl.ds(c0 + 128, 128)]],
                         buf.at[pl.ds(128, 128)], sem)
        pltpu.async_copy(ctab.at[hidx_v.at[pl.ds(c0 + 256, 64)]],
                         buf.at[pl.ds(256, 64)], sem)

    def drain(src, dst, sem):
        pltpu.make_async_copy(src, dst, sem).wait()

    def wait_h(buf, sem):
        drain(hout.at[pl.ds(0, HROWS)], buf, sem)

    def valu(buf, hb):
        def row(r, _):
            rb = r * L
            for col in range(D // 16):
                cs = pl.ds(col * 16, 16)
                acc = buf[rb, cs]
                for l in range(1, L):
                    acc = acc + buf[rb + l, cs]
                hb[r, cs] = acc
            return 0
        lax.fori_loop(0, HC, row, 0)

    hb0 = habuf.at[0]
    hb1 = habuf.at[1]
    buf0 = hbuf.at[0]
    buf1 = hbuf.at[1]
    fire_h(0, buf0, gsem0)
    fire_h(1, buf1, gsem1)

    def pair(i, _):
        a = 2 * i
        b = a + 1

        @pl.when(i > 0)
        def _():
            drain(hout.at[pl.ds(0, HC)], hb0, osem0)
        wait_h(buf0, gsem0)
        valu(buf0, hb0)

        @pl.when(a + 2 < NHC)
        def _():
            fire_h(a + 2, buf0, gsem0)
        pltpu.async_copy(hb0, hout.at[pl.ds(base + a * HC, HC)], osem0)

        @pl.when(i > 0)
        def _():
            drain(hout.at[pl.ds(0, HC)], hb1, osem1)
        wait_h(buf1, gsem1)
        valu(buf1, hb1)

        @pl.when(b + 2 < NHC)
        def _():
            fire_h(b + 2, buf1, gsem1)
        pltpu.async_copy(hb1, hout.at[pl.ds(base + b * HC, HC)], osem1)
        return 0

    lax.fori_loop(0, NHC // 2, pair, 0)
    drain(hout.at[pl.ds(0, HC)], hb0, osem0)
    drain(hout.at[pl.ds(0, HC)], hb1, osem1)


@jax.jit
def _sc_gather(uidx, iidx, hidx, utab, ctab):
    mesh = plsc.VectorSubcoreMesh(core_axis_name="c", subcore_axis_name="s")
    f = pl.kernel(
        _sc_body,
        out_type=[jax.ShapeDtypeStruct((B, D), jnp.float32)] * 3,
        mesh=mesh,
        scratch_types=[
            pltpu.VMEM((RPW,), jnp.int32),
            pltpu.VMEM((RPW,), jnp.int32),
            pltpu.VMEM((RPW * L,), jnp.int32),
            pltpu.VMEM((2, UI_CHUNK, D), jnp.float32),
            pltpu.VMEM((2, HROWS, D), jnp.float32),
            pltpu.VMEM((2, HC, D), jnp.float32),
        ] + [pltpu.SemaphoreType.DMA] * 8,
    )
    return f(uidx, iidx, hidx, utab, ctab)


BLK = 2048


def _stats_body(u_ref, i_ref, h_ref, s_out, q_out, acc_s, acc_q):
    @pl.when(pl.program_id(0) == 0)
    def _():
        acc_s[...] = jnp.zeros_like(acc_s)
        acc_q[...] = jnp.zeros_like(acc_q)

    x = jnp.concatenate([u_ref[...], i_ref[...], h_ref[...]], axis=1)
    acc_s[...] += jnp.sum(x, axis=0, keepdims=True)
    acc_q[...] += jnp.sum(x * x, axis=0, keepdims=True)

    @pl.when(pl.program_id(0) == pl.num_programs(0) - 1)
    def _():
        s_out[...] = acc_s[...]
        q_out[...] = acc_q[...]


def _apply_body(u_ref, i_ref, h_ref, s_ref, q_ref, g_ref, be_ref,
                w1_ref, b1_ref, a1_ref, w2_ref, b2_ref, a2_ref,
                w3_ref, b3_ref, o_ref):
    inv_b = 1.0 / B
    mean = s_ref[...] * inv_b
    var = q_ref[...] * inv_b - mean * mean
    inv = lax.rsqrt(var + 1e-5)
    scale = g_ref[...] * inv
    shift = be_ref[...] - mean * scale
    x = jnp.concatenate([u_ref[...], i_ref[...], h_ref[...]], axis=1)
    xh = x * scale + shift
    a1 = a1_ref[0, 0]
    h1 = jnp.dot(xh, w1_ref[...], preferred_element_type=jnp.float32) + b1_ref[...]
    h1 = jnp.maximum(h1, 0.0) + a1 * jnp.minimum(h1, 0.0)
    a2 = a2_ref[0, 0]
    h2 = jnp.dot(h1, w2_ref[...], preferred_element_type=jnp.float32) + b2_ref[...]
    h2 = jnp.maximum(h2, 0.0) + a2 * jnp.minimum(h2, 0.0)
    lg = jnp.dot(h2, w3_ref[...], preferred_element_type=jnp.float32) + b3_ref[...]
    m = jnp.max(lg, axis=1, keepdims=True)
    e = jnp.exp(lg - m)
    o_ref[...] = e / jnp.sum(e, axis=1, keepdims=True)


def _tc_head(u_rows, i_rows, h_rows, bn_gamma, bn_beta, W1, b1, a1, W2, b2, a2, W3, b3):
    nblk = B // BLK
    blk = lambda j: pl.BlockSpec((BLK, D), lambda i: (i, 0))
    full = lambda r, c: pl.BlockSpec((r, c), lambda i: (0, 0))
    sums, sqs = pl.pallas_call(
        _stats_body,
        grid=(nblk,),
        in_specs=[blk(0), blk(1), blk(2)],
        out_specs=[full(1, 384), full(1, 384)],
        out_shape=[jax.ShapeDtypeStruct((1, 384), jnp.float32)] * 2,
        scratch_shapes=[pltpu.VMEM((1, 384), jnp.float32)] * 2,
    )(u_rows, i_rows, h_rows)

    probs = pl.pallas_call(
        _apply_body,
        grid=(nblk,),
        in_specs=[blk(0), blk(1), blk(2),
                  full(1, 384), full(1, 384), full(1, 384), full(1, 384),
                  full(384, 200), full(1, 200), full(1, 1),
                  full(200, 80), full(1, 80), full(1, 1),
                  full(80, 2), full(1, 2)],
        out_specs=pl.BlockSpec((BLK, 2), lambda i: (i, 0)),
        out_shape=jax.ShapeDtypeStruct((B, 2), jnp.float32),
    )(u_rows, i_rows, h_rows, sums, sqs,
      bn_gamma.reshape(1, 384), bn_beta.reshape(1, 384),
      W1, b1.reshape(1, 200), a1.reshape(1, 1),
      W2, b2.reshape(1, 80), a2.reshape(1, 1),
      W3, b3.reshape(1, 2))
    return probs


def kernel(user, item, history, length, user_table, item_table, cate_table,
           cate_list, bn_gamma, bn_beta, W1, b1, a1, W2, b2, a2, W3, b3):
    ctab = jnp.concatenate(
        [item_table, jnp.take(cate_table, cate_list, axis=0)], axis=1)
    u_rows, i_rows, h_rows = _sc_gather(
        user.astype(jnp.int32), item.astype(jnp.int32),
        history.reshape(-1).astype(jnp.int32), user_table, ctab)
    return _tc_head(u_rows, i_rows, h_rows, bn_gamma, bn_beta,
                    W1, b1, a1, W2, b2, a2, W3, b3)


# speedup vs baseline: 25.3491x; 1.0008x over previous
import jax
import jax.numpy as jnp
from jax import lax
from jax.experimental import pallas as pl
from jax.experimental.pallas import tpu as pltpu
from jax.experimental.pallas import tpu_sc as plsc

B = 16384
L = 20
D = 128
NC = 2
NS = 16
NW = NC * NS
RPW = B // NW
UI_CHUNK = 128
HC = 16
NHC = RPW // HC
HROWS = HC * L


def _sc_body(uidx_hbm, iidx_hbm, hidx_hbm, utab, ctab, uout, iout, hout,
             uidx_v, iidx_v, hidx_v, rbuf, hbuf, habuf,
             gsem0, gsem1, osem0, osem1, usem0, usem1, wsem0, wsem1):
    c = lax.axis_index("c")
    s = lax.axis_index("s")
    wid = c * NS + s
    base = wid * RPW

    with jax.named_scope("idx_stage"):
        cp_u = pltpu.async_copy(uidx_hbm.at[pl.ds(base, RPW)], uidx_v, usem0)
        cp_i = pltpu.async_copy(iidx_hbm.at[pl.ds(base, RPW)], iidx_v, usem1)
        cp_h = pltpu.async_copy(hidx_hbm.at[pl.ds(base * L, RPW * L)], hidx_v, gsem0)
        cp_u.wait()
        cp_i.wait()
        cp_h.wait()

    plan = ([(utab, uidx_v, uout, k) for k in range(RPW // UI_CHUNK)] +
            [(ctab, iidx_v, iout, k) for k in range(RPW // UI_CHUNK)])
    gse = [usem0, usem1]
    wse = [wsem0, wsem1]

    def fire_ui(t):
        tab, idxv, _, k = plan[t]
        return pltpu.async_copy(
            tab.at[idxv.at[pl.ds(k * UI_CHUNK, UI_CHUNK)]], rbuf.at[t % 2],
            gse[t % 2])

    with jax.named_scope("ui_gather"):
        hs = {0: fire_ui(0)}
        outs = {}
        for t in range(len(plan)):
            if t + 1 < len(plan):
                if t - 1 >= 0:
                    outs[t - 1].wait()
                hs[t + 1] = fire_ui(t + 1)
            hs[t].wait()
            _, _, out, k = plan[t]
            outs[t] = pltpu.async_copy(
                rbuf.at[t % 2], out.at[pl.ds(base + k * UI_CHUNK, UI_CHUNK)],
                wse[t % 2])
        outs[len(plan) - 2].wait()
        outs[len(plan) - 1].wait()

    def fire_h(ci, buf, sem):
        c0 = ci * HROWS
        pltpu.async_copy(ctab.at[hidx_v.at[pl.ds(c0, 128)]],
                         buf.at[pl.ds(0, 128)], sem)
        pltpu.async_copy(ctab.at[hidx_v.at[pl.ds(c0 + 128, 128)]],
                         buf.at[pl.ds(128, 128)], sem)
        pltpu.async_copy(ctab.at[hidx_v.at[pl.ds(c0 + 256, 64)]],
                         buf.at[pl.ds(256, 64)], sem)

    def drain(src, dst, sem):
        pltpu.make_async_copy(src, dst, sem).wait()

    def wait_h(buf, sem):
        drain(hout.at[pl.ds(0, HROWS)], buf, sem)

    def valu(buf, hb):
        def row(r, _):
            rb = r * L
            for col in range(D // 16):
                cs = pl.ds(col * 16, 16)
                acc = buf[rb, cs]
                for l in range(1, L):
                    acc = acc + buf[rb + l, cs]
                hb[r, cs] = acc
            return 0
        lax.fori_loop(0, HC, row, 0)

    hb0 = habuf.at[0]
    hb1 = habuf.at[1]
    buf0 = hbuf.at[0]
    buf1 = hbuf.at[1]
    fire_h(0, buf0, gsem0)
    fire_h(1, buf1, gsem1)

    def pair(i, _):
        a = 2 * i
        b = a + 1

        @pl.when(i > 0)
        def _():
            drain(hout.at[pl.ds(0, HC)], hb0, osem0)
        wait_h(buf0, gsem0)
        valu(buf0, hb0)

        @pl.when(a + 2 < NHC)
        def _():
            fire_h(a + 2, buf0, gsem0)
        pltpu.async_copy(hb0, hout.at[pl.ds(base + a * HC, HC)], osem0)

        @pl.when(i > 0)
        def _():
            drain(hout.at[pl.ds(0, HC)], hb1, osem1)
        wait_h(buf1, gsem1)
        valu(buf1, hb1)

        @pl.when(b + 2 < NHC)
        def _():
            fire_h(b + 2, buf1, gsem1)
        pltpu.async_copy(hb1, hout.at[pl.ds(base + b * HC, HC)], osem1)
        return 0

    with jax.named_scope("hist_pool"):
        lax.fori_loop(0, NHC // 2, pair, 0)
        drain(hout.at[pl.ds(0, HC)], hb0, osem0)
        drain(hout.at[pl.ds(0, HC)], hb1, osem1)


@jax.jit
def _sc_gather(uidx, iidx, hidx, utab, ctab):
    mesh = plsc.VectorSubcoreMesh(core_axis_name="c", subcore_axis_name="s")
    f = pl.kernel(
        _sc_body,
        out_type=[jax.ShapeDtypeStruct((B, D), jnp.float32)] * 3,
        mesh=mesh,
        scratch_types=[
            pltpu.VMEM((RPW,), jnp.int32),
            pltpu.VMEM((RPW,), jnp.int32),
            pltpu.VMEM((RPW * L,), jnp.int32),
            pltpu.VMEM((2, UI_CHUNK, D), jnp.float32),
            pltpu.VMEM((2, HROWS, D), jnp.float32),
            pltpu.VMEM((2, HC, D), jnp.float32),
        ] + [pltpu.SemaphoreType.DMA] * 8,
    )
    return f(uidx, iidx, hidx, utab, ctab)


BLK = 2048


def _stats_body(u_ref, i_ref, h_ref, s_out, q_out, acc_s, acc_q):
    @pl.when(pl.program_id(0) == 0)
    def _():
        acc_s[...] = jnp.zeros_like(acc_s)
        acc_q[...] = jnp.zeros_like(acc_q)

    x = jnp.concatenate([u_ref[...], i_ref[...], h_ref[...]], axis=1)
    acc_s[...] += jnp.sum(x, axis=0, keepdims=True)
    acc_q[...] += jnp.sum(x * x, axis=0, keepdims=True)

    @pl.when(pl.program_id(0) == pl.num_programs(0) - 1)
    def _():
        s_out[...] = acc_s[...]
        q_out[...] = acc_q[...]


def _apply_body(u_ref, i_ref, h_ref, s_ref, q_ref, g_ref, be_ref,
                w1_ref, b1_ref, a1_ref, w2_ref, b2_ref, a2_ref,
                w3_ref, b3_ref, o_ref):
    inv_b = 1.0 / B
    mean = s_ref[...] * inv_b
    var = q_ref[...] * inv_b - mean * mean
    inv = lax.rsqrt(var + 1e-5)
    scale = g_ref[...] * inv
    shift = be_ref[...] - mean * scale
    x = jnp.concatenate([u_ref[...], i_ref[...], h_ref[...]], axis=1)
    xh = x * scale + shift
    a1 = a1_ref[0, 0]
    h1 = jnp.dot(xh, w1_ref[...], preferred_element_type=jnp.float32) + b1_ref[...]
    h1 = jnp.maximum(h1, 0.0) + a1 * jnp.minimum(h1, 0.0)
    a2 = a2_ref[0, 0]
    h2 = jnp.dot(h1, w2_ref[...], preferred_element_type=jnp.float32) + b2_ref[...]
    h2 = jnp.maximum(h2, 0.0) + a2 * jnp.minimum(h2, 0.0)
    lg = jnp.dot(h2, w3_ref[...], preferred_element_type=jnp.float32) + b3_ref[...]
    m = jnp.max(lg, axis=1, keepdims=True)
    e = jnp.exp(lg - m)
    o_ref[...] = e / jnp.sum(e, axis=1, keepdims=True)


def _tc_head(u_rows, i_rows, h_rows, bn_gamma, bn_beta, W1, b1, a1, W2, b2, a2, W3, b3):
    nblk = B // BLK
    blk = lambda j: pl.BlockSpec((BLK, D), lambda i: (i, 0))
    full = lambda r, c: pl.BlockSpec((r, c), lambda i: (0, 0))
    sums, sqs = pl.pallas_call(
        _stats_body,
        grid=(nblk,),
        in_specs=[blk(0), blk(1), blk(2)],
        out_specs=[full(1, 384), full(1, 384)],
        out_shape=[jax.ShapeDtypeStruct((1, 384), jnp.float32)] * 2,
        scratch_shapes=[pltpu.VMEM((1, 384), jnp.float32)] * 2,
    )(u_rows, i_rows, h_rows)

    probs = pl.pallas_call(
        _apply_body,
        grid=(nblk,),
        in_specs=[blk(0), blk(1), blk(2),
                  full(1, 384), full(1, 384), full(1, 384), full(1, 384),
                  full(384, 200), full(1, 200), full(1, 1),
                  full(200, 80), full(1, 80), full(1, 1),
                  full(80, 2), full(1, 2)],
        out_specs=pl.BlockSpec((BLK, 2), lambda i: (i, 0)),
        out_shape=jax.ShapeDtypeStruct((B, 2), jnp.float32),
    )(u_rows, i_rows, h_rows, sums, sqs,
      bn_gamma.reshape(1, 384), bn_beta.reshape(1, 384),
      W1, b1.reshape(1, 200), a1.reshape(1, 1),
      W2, b2.reshape(1, 80), a2.reshape(1, 1),
      W3, b3.reshape(1, 2))
    return probs


def kernel(user, item, history, length, user_table, item_table, cate_table,
           cate_list, bn_gamma, bn_beta, W1, b1, a1, W2, b2, a2, W3, b3):
    ctab = jnp.concatenate(
        [item_table, jnp.take(cate_table, cate_list, axis=0)], axis=1)
    u_rows, i_rows, h_rows = _sc_gather(
        user.astype(jnp.int32), item.astype(jnp.int32),
        history.reshape(-1).astype(jnp.int32), user_table, ctab)
    return _tc_head(u_rows, i_rows, h_rows, bn_gamma, bn_beta,
                    W1, b1, a1, W2, b2, a2, W3, b3)
